# Initial kernel scaffold; baseline (speedup 1.0000x reference)
import jax
import jax.numpy as jnp
from jax.experimental import pallas as pl


def kernel(feature_table, nodes, neighbor_idx, W, b):
    raise NotImplementedError("write your pallas kernel here")


# speedup vs baseline: 3.4541x; 3.4541x over previous
import functools

import jax
import jax.numpy as jnp
from jax import lax
from jax.experimental import pallas as pl
from jax.experimental.pallas import tpu as pltpu
from jax.experimental.pallas import tpu_sc as plsc

_B = 16384
_D = 256
_S = 10
_E = 256

_INFO = plsc.get_sparse_core_info()
_NC = _INFO.num_cores
_NS = _INFO.num_subcores
_NW = _NC * _NS
_BPW = _B // _NW

_NCH = 64
_CB = 8


def _sc_body(table, nodes_idx, nbr_idx, out_nodes, out_mean,
             idxn_v, idxb_v, nbuf, bbuf, mbuf, sem):
    wid = lax.axis_index("s") * _NC + lax.axis_index("c")
    base = pl.multiple_of(wid * _BPW, _BPW)

    pltpu.sync_copy(nodes_idx.at[pl.ds(base, _BPW)], idxn_v)
    pltpu.sync_copy(nbr_idx.at[pl.ds(base * _S, _BPW * _S)], idxb_v)

    def chunk_a(g, _):
        off = pl.multiple_of(g * _NCH, _NCH)
        pltpu.async_copy(table.at[idxn_v.at[pl.ds(off, _NCH)]], nbuf, sem).wait()
        pltpu.sync_copy(nbuf, out_nodes.at[pl.ds(base + off, _NCH)])
        return _
    lax.fori_loop(0, _BPW // _NCH, chunk_a, None)

    inv_s = jnp.float32(1.0 / _S)

    def chunk_b(g, _):
        off = pl.multiple_of(g * _CB, _CB)
        pltpu.async_copy(
            table.at[idxb_v.at[pl.ds(off * _S, _CB * _S)]], bbuf, sem).wait()

        def row(i, _):
            r0 = i * _S
            for d in range(_D // 16):
                sl = pl.ds(d * 16, 16)
                acc = bbuf[r0, sl]
                for s in range(1, _S):
                    acc = acc + bbuf[r0 + s, sl]
                mbuf[i, sl] = acc * inv_s
            return _
        lax.fori_loop(0, _CB, row, None)
        pltpu.sync_copy(mbuf, out_mean.at[pl.ds(base + off, _CB)])
        return _
    lax.fori_loop(0, _BPW // _CB, chunk_b, None)


_sc_gather = pl.kernel(
    _sc_body,
    out_type=(
        jax.ShapeDtypeStruct((_B, _D), jnp.float32),
        jax.ShapeDtypeStruct((_B, _D), jnp.float32),
    ),
    mesh=plsc.VectorSubcoreMesh(core_axis_name="c", subcore_axis_name="s"),
    scratch_types=[
        pltpu.VMEM((_BPW,), jnp.int32),
        pltpu.VMEM((_BPW * _S,), jnp.int32),
        pltpu.VMEM((_NCH, _D), jnp.float32),
        pltpu.VMEM((_CB * _S, _D), jnp.float32),
        pltpu.VMEM((_CB, _D), jnp.float32),
        pltpu.SemaphoreType.DMA,
    ],
)


def _tc_body(an_ref, am_ref, w1_ref, w2_ref, b_ref, o_ref):
    acc = jnp.dot(an_ref[...], w1_ref[...], preferred_element_type=jnp.float32)
    acc = acc + jnp.dot(am_ref[...], w2_ref[...],
                        preferred_element_type=jnp.float32)
    o_ref[...] = jnp.maximum(acc + b_ref[...], 0.0)


_BM = 1024


@functools.partial(jax.jit, static_argnums=())
def _tc_matmul(an, am, w1, w2, b2):
    return pl.pallas_call(
        _tc_body,
        grid=(_B // _BM,),
        in_specs=[
            pl.BlockSpec((_BM, _D), lambda i: (i, 0)),
            pl.BlockSpec((_BM, _D), lambda i: (i, 0)),
            pl.BlockSpec((_D, _E), lambda i: (0, 0)),
            pl.BlockSpec((_D, _E), lambda i: (0, 0)),
            pl.BlockSpec((1, _E), lambda i: (0, 0)),
        ],
        out_specs=pl.BlockSpec((_BM, _E), lambda i: (i, 0)),
        out_shape=jax.ShapeDtypeStruct((_B, _E), jnp.float32),
    )(an, am, w1, w2, b2)


def kernel(feature_table, nodes, neighbor_idx, W, b):
    nodes_i = nodes.astype(jnp.int32)
    nbr_i = neighbor_idx.astype(jnp.int32).reshape(-1)
    nodes_feat, neigh_mean = _sc_gather(feature_table, nodes_i, nbr_i)
    return _tc_matmul(nodes_feat, neigh_mean, W[:_D], W[_D:],
                      b.reshape(1, _E))


# speedup vs baseline: 4.4740x; 1.2953x over previous
import functools

import jax
import jax.numpy as jnp
from jax import lax
from jax.experimental import pallas as pl
from jax.experimental.pallas import tpu as pltpu
from jax.experimental.pallas import tpu_sc as plsc

_B = 16384
_D = 256
_S = 10
_E = 256
_R = _S + 1

_INFO = plsc.get_sparse_core_info()
_NC = _INFO.num_cores
_NS = _INFO.num_subcores
_NW = _NC * _NS
_BPW = _B // _NW

_CB = 8
_G = _BPW // _CB
_NBUF = 4
_T = _G // _NBUF

_NV = _D // 16


def _sc_body(table, cidx, out_nodes, out_mean,
             idx_v, bbuf, mbuf, nstage,
             sem_g0, sem_g1, sem_g2, sem_g3, sem_o):
    sems_g = (sem_g0, sem_g1, sem_g2, sem_g3)
    wid = lax.axis_index("s") * _NC + lax.axis_index("c")
    base = pl.multiple_of(wid * _BPW, _BPW)

    pltpu.sync_copy(cidx.at[pl.ds(base * _R, _BPW * _R)], idx_v)

    def gather_src(g):
        off = pl.multiple_of(g * (_CB * _R), _CB * _R)
        return table.at[idx_v.at[pl.ds(off, _CB * _R)]]

    def fire(g, b):
        pltpu.async_copy(gather_src(g), bbuf.at[b], sems_g[b])

    def process(g, b, fire_next):
        pltpu.make_async_copy(gather_src(g), bbuf.at[b], sems_g[b]).wait()

        def row(i, _):
            r0 = i * _R
            for d in range(_NV):
                sl = pl.ds(d * 16, 16)
                nstage[i, sl] = bbuf[b, r0, sl]
                acc = bbuf[b, r0 + 1, sl]
                for s in range(2, _R):
                    acc = acc + bbuf[b, r0 + s, sl]
                mbuf[i, sl] = acc * jnp.float32(1.0 / _S)
            return _
        lax.fori_loop(0, _CB, row, None)

        if fire_next is not None:
            fire(fire_next, b)

        orow = base + g * _CB
        cp_n = pltpu.async_copy(nstage, out_nodes.at[pl.ds(orow, _CB)], sem_o)
        cp_m = pltpu.async_copy(mbuf, out_mean.at[pl.ds(orow, _CB)], sem_o)
        cp_n.wait()
        cp_m.wait()

    for b in range(_NBUF):
        fire(b, b)

    def outer(t, _):
        g0 = t * _NBUF
        for b in range(_NBUF):
            process(g0 + b, b, g0 + b + _NBUF)
        return _
    lax.fori_loop(0, _T - 1, outer, None)

    for b in range(_NBUF):
        process((_T - 1) * _NBUF + b, b, None)


_sc_gather = pl.kernel(
    _sc_body,
    out_type=(
        jax.ShapeDtypeStruct((_B, _D), jnp.float32),
        jax.ShapeDtypeStruct((_B, _D), jnp.float32),
    ),
    mesh=plsc.VectorSubcoreMesh(core_axis_name="c", subcore_axis_name="s"),
    scratch_types=[
        pltpu.VMEM((_BPW * _R,), jnp.int32),
        pltpu.VMEM((_NBUF, _CB * _R, _D), jnp.float32),
        pltpu.VMEM((_CB, _D), jnp.float32),
        pltpu.VMEM((_CB, _D), jnp.float32),
        pltpu.SemaphoreType.DMA,
        pltpu.SemaphoreType.DMA,
        pltpu.SemaphoreType.DMA,
        pltpu.SemaphoreType.DMA,
        pltpu.SemaphoreType.DMA,
    ],
)


def _tc_body(an_ref, am_ref, w1_ref, w2_ref, b_ref, o_ref):
    acc = jnp.dot(an_ref[...], w1_ref[...], preferred_element_type=jnp.float32)
    acc = acc + jnp.dot(am_ref[...], w2_ref[...],
                        preferred_element_type=jnp.float32)
    o_ref[...] = jnp.maximum(acc + b_ref[...], 0.0)


_BM = 1024


def _tc_matmul(an, am, w1, w2, b2):
    return pl.pallas_call(
        _tc_body,
        grid=(_B // _BM,),
        in_specs=[
            pl.BlockSpec((_BM, _D), lambda i: (i, 0)),
            pl.BlockSpec((_BM, _D), lambda i: (i, 0)),
            pl.BlockSpec((_D, _E), lambda i: (0, 0)),
            pl.BlockSpec((_D, _E), lambda i: (0, 0)),
            pl.BlockSpec((1, _E), lambda i: (0, 0)),
        ],
        out_specs=pl.BlockSpec((_BM, _E), lambda i: (i, 0)),
        out_shape=jax.ShapeDtypeStruct((_B, _E), jnp.float32),
    )(an, am, w1, w2, b2)


def kernel(feature_table, nodes, neighbor_idx, W, b):
    nodes_i = nodes.astype(jnp.int32)
    nbr_i = neighbor_idx.astype(jnp.int32)
    cidx = jnp.concatenate([nodes_i[:, None], nbr_i], axis=1).reshape(-1)
    nodes_feat, neigh_mean = _sc_gather(feature_table, cidx)
    return _tc_matmul(nodes_feat, neigh_mean, W[:_D], W[_D:],
                      b.reshape(1, _E))


# speedup vs baseline: 5.0388x; 1.1262x over previous
import jax
import jax.numpy as jnp
from jax import lax
from jax.experimental import pallas as pl
from jax.experimental.pallas import tpu as pltpu
from jax.experimental.pallas import tpu_sc as plsc

_N = 50000
_B = 16384
_D = 256
_S = 10
_E = 256
_R = _S + 1
_DP = _D // 2

_INFO = plsc.get_sparse_core_info()
_NC = _INFO.num_cores
_NS = _INFO.num_subcores
_NW = _NC * _NS
_BPW = _B // _NW

_CB = 8
_G = _BPW // _CB
_NBUF = 4
_T = _G // _NBUF

_NG = _D // 32


_BM_PRE = 2000
_PRE_STEPS = _N // _BM_PRE


def _prepass_body(t_ref, wlo_ref, whi_ref, blo_ref, bhi_ref, o_ref):
    t = t_ref[...].astype(jnp.bfloat16)
    wlo = wlo_ref[0].astype(jnp.bfloat16)
    whi = whi_ref[0].astype(jnp.bfloat16)
    lo = jnp.dot(t, wlo, preferred_element_type=jnp.float32) + blo_ref[0]
    hi = jnp.dot(t, whi, preferred_element_type=jnp.float32) + bhi_ref[0]
    lo_bits = lax.bitcast_convert_type(
        lo.astype(jnp.bfloat16).astype(jnp.float32), jnp.int32)
    hi_bits = lax.bitcast_convert_type(
        hi.astype(jnp.bfloat16).astype(jnp.float32), jnp.int32)
    o_ref[...] = (hi_bits & jnp.int32(-65536)) | (
        lax.shift_right_logical(lo_bits, 16))


def _prepass(table, wlo2, whi2, blo2, bhi2):
    return pl.pallas_call(
        _prepass_body,
        grid=(2 * _PRE_STEPS,),
        in_specs=[
            pl.BlockSpec((_BM_PRE, _D), lambda i: (i % _PRE_STEPS, 0)),
            pl.BlockSpec((1, _D, _DP), lambda i: (i // _PRE_STEPS, 0, 0)),
            pl.BlockSpec((1, _D, _DP), lambda i: (i // _PRE_STEPS, 0, 0)),
            pl.BlockSpec((1, 1, _DP), lambda i: (i // _PRE_STEPS, 0, 0)),
            pl.BlockSpec((1, 1, _DP), lambda i: (i // _PRE_STEPS, 0, 0)),
        ],
        out_specs=pl.BlockSpec((_BM_PRE, _DP), lambda i: (i, 0)),
        out_shape=jax.ShapeDtypeStruct((2 * _N, _DP), jnp.int32),
    )(table, wlo2, whi2, blo2, bhi2)


def _sc_body(tpk, cidx, out_h, idx_v, bbuf, hbuf, sem_g0, sem_g1, sem_g2,
             sem_g3, sem_o):
    sems_g = (sem_g0, sem_g1, sem_g2, sem_g3)
    wid = lax.axis_index("s") * _NC + lax.axis_index("c")
    base = pl.multiple_of(wid * _BPW, _BPW)

    pltpu.sync_copy(cidx.at[pl.ds(base * _R, _BPW * _R)], idx_v)

    def gather_src(g):
        off = pl.multiple_of(g * (_CB * _R), _CB * _R)
        return tpk.at[idx_v.at[pl.ds(off, _CB * _R)]]

    def fire(g, b):
        pltpu.async_copy(gather_src(g), bbuf.at[b], sems_g[b])

    hi_mask = jnp.int32(-65536)
    inv_s = jnp.float32(1.0 / _S)

    def process(g, b, fire_next):
        pltpu.make_async_copy(gather_src(g), bbuf.at[b], sems_g[b]).wait()

        def row(i, _):
            r0 = i * _R
            for d in range(_NG):
                sl = pl.ds(d * 16, 16)
                v = bbuf[b, r0 + 1, sl]
                alo = lax.bitcast_convert_type(v << 16, jnp.float32)
                ahi = lax.bitcast_convert_type(v & hi_mask, jnp.float32)
                for s in range(2, _R):
                    v = bbuf[b, r0 + s, sl]
                    alo = alo + lax.bitcast_convert_type(v << 16, jnp.float32)
                    ahi = ahi + lax.bitcast_convert_type(v & hi_mask, jnp.float32)
                vn = bbuf[b, r0, sl]
                alo = alo * inv_s + lax.bitcast_convert_type(vn << 16, jnp.float32)
                ahi = ahi * inv_s + lax.bitcast_convert_type(vn & hi_mask, jnp.float32)
                hbuf[i, pl.ds(d * 32, 16)] = jnp.maximum(alo, 0.0)
                hbuf[i, pl.ds(d * 32 + 16, 16)] = jnp.maximum(ahi, 0.0)
            return _
        lax.fori_loop(0, _CB, row, None)

        if fire_next is not None:
            fire(fire_next, b)

        cp = pltpu.async_copy(hbuf, out_h.at[pl.ds(base + g * _CB, _CB)],
                              sem_o)
        cp.wait()

    for b in range(_NBUF):
        fire(b, b)

    def outer(t, _):
        g0 = t * _NBUF
        for b in range(_NBUF):
            process(g0 + b, b, g0 + b + _NBUF)
        return _
    lax.fori_loop(0, _T - 1, outer, None)

    for b in range(_NBUF):
        process((_T - 1) * _NBUF + b, b, None)


_sc_encode = pl.kernel(
    _sc_body,
    out_type=jax.ShapeDtypeStruct((_B, _E), jnp.float32),
    mesh=plsc.VectorSubcoreMesh(core_axis_name="c", subcore_axis_name="s"),
    scratch_types=[
        pltpu.VMEM((_BPW * _R,), jnp.int32),
        pltpu.VMEM((_NBUF, _CB * _R, _DP), jnp.int32),
        pltpu.VMEM((_CB, _E), jnp.float32),
        pltpu.SemaphoreType.DMA,
        pltpu.SemaphoreType.DMA,
        pltpu.SemaphoreType.DMA,
        pltpu.SemaphoreType.DMA,
        pltpu.SemaphoreType.DMA,
    ],
)


def kernel(feature_table, nodes, neighbor_idx, W, b):
    w3 = W.reshape(2 * _D, _NG, 32)
    wlo = w3[:, :, :16].reshape(2 * _D, _DP)
    whi = w3[:, :, 16:].reshape(2 * _D, _DP)
    b3 = b.reshape(_NG, 32)
    blo = b3[:, :16].reshape(1, _DP)
    bhi = b3[:, 16:].reshape(1, _DP)
    wlo2 = jnp.stack([wlo[:_D], wlo[_D:]])
    whi2 = jnp.stack([whi[:_D], whi[_D:]])
    blo2 = jnp.stack([blo, jnp.zeros_like(blo)])
    bhi2 = jnp.stack([bhi, jnp.zeros_like(bhi)])

    tpk = _prepass(feature_table, wlo2, whi2, blo2, bhi2)

    nodes_i = nodes.astype(jnp.int32)
    nbr_i = neighbor_idx.astype(jnp.int32) + jnp.int32(_N)
    cidx = jnp.concatenate([nodes_i[:, None], nbr_i], axis=1).reshape(-1)
    return _sc_encode(tpk, cidx)
